# Initial kernel scaffold; baseline (speedup 1.0000x reference)
import jax
import jax.numpy as jnp
from jax.experimental import pallas as pl


def kernel(query, positive_key):
    raise NotImplementedError("write your pallas kernel here")


# speedup vs baseline: 15.7348x; 15.7348x over previous
import functools

import jax
import jax.numpy as jnp
from jax.experimental import pallas as pl

N = 4096
D = 128
TEMP = 0.1
K = max(1, int(0.5 * (N - 1)))
BLOCK = 512
N_ITERS = 32
LO0 = -1.1
HI0 = 1.1


def _loss_block_kernel(q_ref, p_ref, out_ref):
    blk = pl.program_id(0)

    qb = q_ref[...]
    pf = p_ref[...]

    qn = qb / jnp.maximum(jnp.sqrt(jnp.sum(qb * qb, axis=1, keepdims=True)), 1e-12)
    pn = pf / jnp.maximum(jnp.sqrt(jnp.sum(pf * pf, axis=1, keepdims=True)), 1e-12)

    logits = jax.lax.dot_general(
        qn, pn, dimension_numbers=(((1,), (1,)), ((), ())),
        preferred_element_type=jnp.float32,
    )

    col_ids = jax.lax.broadcasted_iota(jnp.int32, (BLOCK, N), 1)
    row_ids = blk * BLOCK + jax.lax.broadcasted_iota(jnp.int32, (BLOCK, N), 0)
    diag = col_ids == row_ids

    pos = jnp.sum(jnp.where(diag, logits, 0.0), axis=1, keepdims=True)
    masked = jnp.where(diag, 2.0, logits)

    kf = jnp.float32(K)

    def bs_body(_, carry):
        lo, hi = carry
        mid = 0.5 * (lo + hi)
        cnt = jnp.sum((masked < mid).astype(jnp.float32), axis=1, keepdims=True)
        ge = cnt >= kf
        return jnp.where(ge, lo, mid), jnp.where(ge, mid, hi)

    lo = jnp.full((BLOCK, 1), LO0, jnp.float32)
    hi = jnp.full((BLOCK, 1), HI0, jnp.float32)
    lo, hi = jax.lax.fori_loop(0, N_ITERS, bs_body, (lo, hi))
    t = 0.5 * (lo + hi)

    m = jnp.maximum(pos, t) / TEMP
    below = masked < t
    ex = jnp.where(below, jnp.exp(masked / TEMP - m), 0.0)
    cnt = jnp.sum(below.astype(jnp.float32), axis=1, keepdims=True)
    s = jnp.sum(ex, axis=1, keepdims=True)
    s = s + (kf - cnt) * jnp.exp(t / TEMP - m) + jnp.exp(pos / TEMP - m)
    losses = -pos / TEMP + m + jnp.log(s)

    block_sum = jnp.sum(losses).reshape(1, 1)

    @pl.when(blk == 0)
    def _init():
        out_ref[...] = block_sum

    @pl.when(blk != 0)
    def _acc():
        out_ref[...] += block_sum


@jax.jit
def kernel(query, positive_key):
    out = pl.pallas_call(
        _loss_block_kernel,
        grid=(N // BLOCK,),
        in_specs=[
            pl.BlockSpec((BLOCK, D), lambda i: (i, 0)),
            pl.BlockSpec((N, D), lambda i: (0, 0)),
        ],
        out_specs=pl.BlockSpec((1, 1), lambda i: (0, 0)),
        out_shape=jax.ShapeDtypeStruct((1, 1), jnp.float32),
    )(query, positive_key)
    return out[0, 0] / N


# speedup vs baseline: 22.5633x; 1.4340x over previous
import functools

import jax
import jax.numpy as jnp
from jax.experimental import pallas as pl

N = 4096
D = 128
TEMP = 0.1
K = max(1, int(0.5 * (N - 1)))
BLOCK = 1024
N_ITERS = 22
LO0 = -1.1
HI0 = 1.1


def _loss_block_kernel(q_ref, p_ref, out_ref):
    blk = pl.program_id(0)

    qb = q_ref[...]
    pf = p_ref[...]

    qn = qb / jnp.maximum(jnp.sqrt(jnp.sum(qb * qb, axis=1, keepdims=True)), 1e-12)
    pn = pf / jnp.maximum(jnp.sqrt(jnp.sum(pf * pf, axis=1, keepdims=True)), 1e-12)

    logits = jax.lax.dot_general(
        qn, pn, dimension_numbers=(((1,), (1,)), ((), ())),
        preferred_element_type=jnp.float32,
    )

    col_ids = jax.lax.broadcasted_iota(jnp.int32, (BLOCK, N), 1)
    row_ids = blk * BLOCK + jax.lax.broadcasted_iota(jnp.int32, (BLOCK, N), 0)
    diag = col_ids == row_ids

    pos = jnp.sum(jnp.where(diag, logits, 0.0), axis=1, keepdims=True)
    masked = jnp.where(diag, 2.0, logits)

    kf = jnp.float32(K)

    def bs_body(_, carry):
        lo, hi = carry
        mid = 0.5 * (lo + hi)
        cnt = jnp.sum((masked < mid).astype(jnp.float32), axis=1, keepdims=True)
        ge = cnt >= kf
        return jnp.where(ge, lo, mid), jnp.where(ge, mid, hi)

    lo = jnp.full((BLOCK, 1), LO0, jnp.float32)
    hi = jnp.full((BLOCK, 1), HI0, jnp.float32)
    lo, hi = jax.lax.fori_loop(0, N_ITERS, bs_body, (lo, hi))
    t = 0.5 * (lo + hi)

    m = jnp.maximum(pos, t) / TEMP
    below = masked < t
    ex = jnp.where(below, jnp.exp(masked / TEMP - m), 0.0)
    cnt = jnp.sum(below.astype(jnp.float32), axis=1, keepdims=True)
    s = jnp.sum(ex, axis=1, keepdims=True)
    s = s + (kf - cnt) * jnp.exp(t / TEMP - m) + jnp.exp(pos / TEMP - m)
    losses = -pos / TEMP + m + jnp.log(s)

    block_sum = jnp.sum(losses).reshape(1, 1)

    @pl.when(blk == 0)
    def _init():
        out_ref[...] = block_sum

    @pl.when(blk != 0)
    def _acc():
        out_ref[...] += block_sum


@jax.jit
def kernel(query, positive_key):
    out = pl.pallas_call(
        _loss_block_kernel,
        grid=(N // BLOCK,),
        in_specs=[
            pl.BlockSpec((BLOCK, D), lambda i: (i, 0)),
            pl.BlockSpec((N, D), lambda i: (0, 0)),
        ],
        out_specs=pl.BlockSpec((1, 1), lambda i: (0, 0)),
        out_shape=jax.ShapeDtypeStruct((1, 1), jnp.float32),
    )(query, positive_key)
    return out[0, 0] / N


# speedup vs baseline: 23.7723x; 1.0536x over previous
import functools

import jax
import jax.numpy as jnp
from jax.experimental import pallas as pl

N = 4096
D = 128
TEMP = 0.1
K = max(1, int(0.5 * (N - 1)))
BLOCK = 1024
N_ITERS = 22
LO0 = -1.1
HI0 = 1.1


def _loss_block_kernel(q_ref, p_ref, pblk_ref, out_ref):
    blk = pl.program_id(0)

    qb = q_ref[...]
    pf = p_ref[...]

    qn = qb / jnp.maximum(jnp.sqrt(jnp.sum(qb * qb, axis=1, keepdims=True)), 1e-12)
    pn = pf / jnp.maximum(jnp.sqrt(jnp.sum(pf * pf, axis=1, keepdims=True)), 1e-12)

    logits = jax.lax.dot_general(
        qn, pn, dimension_numbers=(((1,), (1,)), ((), ())),
        preferred_element_type=jnp.float32,
    )

    col_ids = jax.lax.broadcasted_iota(jnp.int32, (BLOCK, N), 1)
    row_ids = blk * BLOCK + jax.lax.broadcasted_iota(jnp.int32, (BLOCK, N), 0)
    diag = col_ids == row_ids

    pb = pblk_ref[...]
    pb = pb / jnp.maximum(jnp.sqrt(jnp.sum(pb * pb, axis=1, keepdims=True)), 1e-12)
    pos = jnp.sum(qn * pb, axis=1, keepdims=True)
    masked = jnp.where(diag, 2.0, logits)

    kf = jnp.float32(K)

    def bs_body(_, carry):
        lo, hi = carry
        mid = 0.5 * (lo + hi)
        cnt = jnp.sum((masked < mid).astype(jnp.float32), axis=1, keepdims=True)
        ge = cnt >= kf
        return jnp.where(ge, lo, mid), jnp.where(ge, mid, hi)

    lo = jnp.full((BLOCK, 1), LO0, jnp.float32)
    hi = jnp.full((BLOCK, 1), HI0, jnp.float32)
    lo, hi = jax.lax.fori_loop(0, N_ITERS, bs_body, (lo, hi))
    t = 0.5 * (lo + hi)

    m = jnp.maximum(pos, t) / TEMP
    ex = jnp.exp(jnp.minimum(masked, t) / TEMP - m)
    s = jnp.sum(ex, axis=1, keepdims=True)
    s = s - (N - K) * jnp.exp(t / TEMP - m) + jnp.exp(pos / TEMP - m)
    losses = -pos / TEMP + m + jnp.log(s)

    block_sum = jnp.sum(losses).reshape(1, 1)

    @pl.when(blk == 0)
    def _init():
        out_ref[...] = block_sum

    @pl.when(blk != 0)
    def _acc():
        out_ref[...] += block_sum


@jax.jit
def kernel(query, positive_key):
    out = pl.pallas_call(
        _loss_block_kernel,
        grid=(N // BLOCK,),
        in_specs=[
            pl.BlockSpec((BLOCK, D), lambda i: (i, 0)),
            pl.BlockSpec((N, D), lambda i: (0, 0)),
            pl.BlockSpec((BLOCK, D), lambda i: (i, 0)),
        ],
        out_specs=pl.BlockSpec((1, 1), lambda i: (0, 0)),
        out_shape=jax.ShapeDtypeStruct((1, 1), jnp.float32),
    )(query, positive_key, positive_key)
    return out[0, 0] / N


# speedup vs baseline: 30.7063x; 1.2917x over previous
import functools

import jax
import jax.numpy as jnp
from jax.experimental import pallas as pl

N = 4096
D = 128
TEMP = 0.1
K = max(1, int(0.5 * (N - 1)))
BLOCK = 1024
N_ITERS = 16
LO0 = -1.1
HI0 = 1.1


def _loss_block_kernel(q_ref, p_ref, pblk_ref, out_ref):
    blk = pl.program_id(0)

    qb = q_ref[...]
    pf = p_ref[...]

    qn = qb / jnp.maximum(jnp.sqrt(jnp.sum(qb * qb, axis=1, keepdims=True)), 1e-12)
    pn = pf / jnp.maximum(jnp.sqrt(jnp.sum(pf * pf, axis=1, keepdims=True)), 1e-12)

    logits = jax.lax.dot_general(
        qn, pn, dimension_numbers=(((1,), (1,)), ((), ())),
        preferred_element_type=jnp.float32,
    )

    pb = pblk_ref[...]
    pb = pb / jnp.maximum(jnp.sqrt(jnp.sum(pb * pb, axis=1, keepdims=True)), 1e-12)
    pos = jnp.sum(qn * pb, axis=1, keepdims=True)

    kf = jnp.float32(K)

    def bs_body(_, carry):
        lo, hi = carry
        mid = 0.5 * (lo + hi)
        cnt = jnp.sum((logits < mid).astype(jnp.float32), axis=1, keepdims=True)
        cnt = cnt - (pos < mid).astype(jnp.float32)
        ge = cnt >= kf
        return jnp.where(ge, lo, mid), jnp.where(ge, mid, hi)

    lo = jnp.full((BLOCK, 1), LO0, jnp.float32)
    hi = jnp.full((BLOCK, 1), HI0, jnp.float32)
    lo, hi = jax.lax.fori_loop(0, N_ITERS, bs_body, (lo, hi))
    t = 0.5 * (lo + hi)

    m = jnp.maximum(pos, t) / TEMP
    ex = jnp.exp(jnp.minimum(logits, t) / TEMP - m)
    s = jnp.sum(ex, axis=1, keepdims=True)
    s = (s - jnp.exp(jnp.minimum(pos, t) / TEMP - m)
         - (N - 1 - K) * jnp.exp(t / TEMP - m) + jnp.exp(pos / TEMP - m))
    losses = -pos / TEMP + m + jnp.log(s)

    block_sum = jnp.sum(losses).reshape(1, 1)

    @pl.when(blk == 0)
    def _init():
        out_ref[...] = block_sum

    @pl.when(blk != 0)
    def _acc():
        out_ref[...] += block_sum


@jax.jit
def kernel(query, positive_key):
    out = pl.pallas_call(
        _loss_block_kernel,
        grid=(N // BLOCK,),
        in_specs=[
            pl.BlockSpec((BLOCK, D), lambda i: (i, 0)),
            pl.BlockSpec((N, D), lambda i: (0, 0)),
            pl.BlockSpec((BLOCK, D), lambda i: (i, 0)),
        ],
        out_specs=pl.BlockSpec((1, 1), lambda i: (0, 0)),
        out_shape=jax.ShapeDtypeStruct((1, 1), jnp.float32),
    )(query, positive_key, positive_key)
    return out[0, 0] / N
